# baseline (speedup 1.0000x reference)
import functools

import jax
import jax.numpy as jnp
from jax import lax
from jax.experimental import pallas as pl
from jax.experimental.pallas import tpu as pltpu
from jax.experimental.pallas import tpu_sc as plsc

_VOCAB = 1000000
_C_SIZE = 1000
_EMBED = 64
_B = 16384
_L = 50

_NC = 2
_NS = 16
_NW = _NC * _NS

_N = _B * _L
_PER_W = _N // _NW
_BPW = _B // _NW
_NBH = _BPW // 128
_UNITS = _NBH * _L
_TP = 129

_mesh = plsc.VectorSubcoreMesh(core_axis_name="c", subcore_axis_name="s")


@functools.partial(
    pl.kernel,
    mesh=_mesh,
    compiler_params=pltpu.CompilerParams(
        use_tc_tiling_on_sc=False, needs_layout_passes=False),
    out_type=[
        jax.ShapeDtypeStruct((_L, 8, _B // 128, 8, 128), jnp.float32),
        jax.ShapeDtypeStruct((8, _B // 128, 8, 128), jnp.float32),
    ],
    scratch_types=[
        pltpu.VMEM((_PER_W,), jnp.int32),
        pltpu.VMEM((_BPW,), jnp.int32),
        pltpu.VMEM((128,), jnp.int32),
        pltpu.VMEM((128,), jnp.int32),
        pltpu.VMEM((128, _EMBED), jnp.float32),
        pltpu.VMEM((128, _EMBED), jnp.float32),
        pltpu.VMEM((_EMBED, _TP), jnp.float32),
        pltpu.VMEM((_EMBED, _TP), jnp.float32),
        pltpu.SemaphoreType.DMA,
        pltpu.SemaphoreType.DMA,
        pltpu.SemaphoreType.DMA,
        pltpu.SemaphoreType.DMA,
    ],
)
def _encode(x_hbm, c_hbm, s_tab, c_tab, out_s, out_c,
            x_v, c_v, idx0, idx1, rows0, rows1, t0, t1, g0, g1, s0, s1):
    wid = lax.axis_index("s") * _NC + lax.axis_index("c")
    idxs = (idx0, idx1)
    rows = (rows0, rows1)
    ts = (t0, t1)
    gsems = (g0, g1)
    ssems = (s0, s1)

    iota = jax.lax.iota(jnp.int32, 16)
    iota50 = iota * _L
    e_rows = [iota + 16 * e0 for e0 in range(4)]

    pltpu.sync_copy(x_hbm.at[pl.ds(wid * _PER_W, _PER_W)], x_v)
    pltpu.sync_copy(c_hbm.at[pl.ds(wid * _BPW, _BPW)], c_v)

    def unit_coords(u):
        bl = u // _L
        l = u % _L
        return bl, l

    def build_idx(u, p):
        bl, l = unit_coords(u)
        base = 6400 * bl + l
        for j0 in range(8):
            v = plsc.load_gather(x_v, [iota50 + (base + 800 * j0)])
            idxs[p][pl.ds(16 * j0, 16)] = v

    def fire_gather(u, p):
        pltpu.async_copy(s_tab.at[idxs[p]], rows[p], gsems[p])

    def wait_gather(u, p):
        pltpu.make_async_copy(s_tab.at[idxs[p]], rows[p], gsems[p]).wait()

    def transpose(p):
        rp = rows[p]
        tp = ts[p]

        @pl.loop(0, 128, unroll=8)
        def _t(b):
            col = jnp.full((16,), b, dtype=jnp.int32)
            for e0 in range(4):
                v = rp[b, pl.ds(16 * e0, 16)]
                plsc.store_scatter(tp, [e_rows[e0], col], v)

    def tile_dsts(u):
        bl, l = unit_coords(u)
        bh = _NBH * wid + bl
        return [out_s.at[l, eh, bh] for eh in range(8)]

    def fire_stores(u, p):
        for eh, dst in enumerate(tile_dsts(u)):
            pltpu.async_copy(ts[p].at[pl.ds(8 * eh, 8), pl.ds(0, 128)], dst,
                             ssems[p])

    def wait_stores(u, p):
        for eh, dst in enumerate(tile_dsts(u)):
            pltpu.make_async_copy(
                ts[p].at[pl.ds(8 * eh, 8), pl.ds(0, 128)], dst,
                ssems[p]).wait()

    build_idx(0, 0)
    fire_gather(0, 0)
    build_idx(1, 1)
    fire_gather(1, 1)

    @pl.loop(0, _UNITS, step=2)
    def _units(g):
        for p in range(2):
            u = g + p
            wait_gather(u, p)

            @pl.when(u >= 2)
            def _():
                wait_stores(u - 2, p)

            transpose(p)
            fire_stores(u, p)

            @pl.when(u + 2 < _UNITS)
            def _():
                build_idx(u + 2, p)
                fire_gather(u + 2, p)

    wait_stores(_UNITS - 2, 0)
    wait_stores(_UNITS - 1, 1)

    @pl.loop(0, _NBH)
    def _cunits(bl):
        pltpu.async_copy(c_tab.at[c_v.at[pl.ds(128 * bl, 128)]], rows0,
                         g0).wait()
        transpose(0)
        bh = _NBH * wid + bl
        for eh in range(8):
            pltpu.async_copy(t0.at[pl.ds(8 * eh, 8), pl.ds(0, 128)],
                             out_c.at[eh, bh], s0)
        for eh in range(8):
            pltpu.make_async_copy(t0.at[pl.ds(8 * eh, 8), pl.ds(0, 128)],
                                  out_c.at[eh, bh], s0).wait()


def kernel(inputs_x, inputs_c, s_table, c_table):
    x_flat = inputs_x.reshape(_N)
    out_s5, out_c4 = _encode(x_flat, inputs_c, s_table, c_table)
    out_s = out_s5.transpose(2, 4, 0, 1, 3).reshape(_B, _L, _EMBED)
    out_c = out_c4.transpose(1, 3, 0, 2).reshape(_B, _EMBED)
    return out_s, out_c
